# Initial kernel scaffold; baseline (speedup 1.0000x reference)
import jax
import jax.numpy as jnp
from jax.experimental import pallas as pl


def kernel(x, Wq, Wk, Wv, Wo, alpha_raw, beta_raw):
    raise NotImplementedError("write your pallas kernel here")


# speedup vs baseline: 3.7755x; 3.7755x over previous
import jax
import jax.numpy as jnp
from jax.experimental import pallas as pl
from jax.experimental.pallas import tpu as pltpu

_H = 16
_C = 64
_DH = 64
_LH = 4
_SCALE = 0.125


def _proj_kernel(x_ref, wq_ref, wk_ref, wv_ref, q_ref, k_ref, v_ref):
    xt = x_ref[...]
    q_ref[...] = jnp.dot(xt, wq_ref[...], preferred_element_type=jnp.float32)
    k_ref[...] = jnp.dot(xt, wk_ref[...], preferred_element_type=jnp.float32)
    v_ref[...] = jnp.dot(xt, wv_ref[...], preferred_element_type=jnp.float32)


def _core_kernel(ab_ref, q_ref, k_ref, v_ref, o_ref, m_scr):
    C, DH = _C, _DH
    row = jax.lax.broadcasted_iota(jnp.int32, (C, C), 0)
    col = jax.lax.broadcasted_iota(jnp.int32, (C, C), 1)
    upper = col > row
    eye = jnp.where(row == col, 1.0, 0.0).astype(jnp.float32)

    ab = jax.nn.sigmoid(ab_ref[...])

    m_scr[...] = jnp.zeros_like(m_scr)

    def chunk_body(n, carry):
        r0 = pl.multiple_of(n * C, C)
        rows = pl.ds(r0, C)
        for lh in range(_LH):
            cols = slice(lh * DH, (lh + 1) * DH)
            alpha = ab[0, 0, lh] * 0.5
            beta = ab[0, 1, lh] * 2.0

            qi = q_ref[rows, cols]
            ki = k_ref[rows, cols]
            vi = v_ref[rows, cols]
            nrm = jnp.sqrt(jnp.sum(ki * ki, axis=1, keepdims=True))
            ki = ki / jnp.maximum(nrm, 1e-12)

            M = m_scr[lh]

            scores = jax.lax.dot_general(
                qi, ki, (((1,), (1,)), ((), ())),
                preferred_element_type=jnp.float32) * _SCALE
            scores = jnp.where(upper, -1e30, scores)
            smax = jnp.max(scores, axis=1, keepdims=True)
            e = jnp.exp(scores - smax)
            attn = e / jnp.sum(e, axis=1, keepdims=True)
            attn_out = jnp.dot(attn, vi, preferred_element_type=jnp.float32)

            mem_out = jnp.dot(qi, M, preferred_element_type=jnp.float32)
            o_ref[rows, cols] = attn_out + 0.1 * mem_out

            G = jax.lax.dot_general(
                ki, ki, (((1,), (1,)), ((), ())),
                preferred_element_type=jnp.float32)
            aU = jnp.where(upper, alpha * G, 0.0)
            inv = eye - aU
            npow = aU
            for _ in range(5):
                npow = jnp.dot(npow, npow, preferred_element_type=jnp.float32)
                inv = inv + jnp.dot(inv, npow,
                                    preferred_element_type=jnp.float32)
            R = jnp.dot(inv, ki, preferred_element_type=jnp.float32)
            MKt = jax.lax.dot_general(
                M, ki, (((1,), (1,)), ((), ())),
                preferred_element_type=jnp.float32)
            MKtR = jnp.dot(MKt, R, preferred_element_type=jnp.float32)
            VtR = jax.lax.dot_general(
                vi, R, (((0,), (0,)), ((), ())),
                preferred_element_type=jnp.float32)
            m_scr[lh] = M - alpha * MKtR + beta * VtR
        return carry

    jax.lax.fori_loop(0, 4096 // C, chunk_body, 0)


def _oproj_kernel(y_ref, w_ref, o_ref):
    o_ref[...] = jnp.dot(y_ref[...], w_ref[...],
                         preferred_element_type=jnp.float32)


def kernel(x, Wq, Wk, Wv, Wo, alpha_raw, beta_raw):
    B, T, D = x.shape
    H, DH = _H, _DH
    x2 = x.reshape(B * T, D)
    ngrp = H // _LH
    ab = jnp.stack([alpha_raw.reshape(ngrp, _LH),
                    beta_raw.reshape(ngrp, _LH)], axis=1)

    RT = 512
    nrt = (B * T) // RT
    q2, k2, v2 = pl.pallas_call(
        _proj_kernel,
        grid=(nrt,),
        in_specs=[
            pl.BlockSpec((RT, D), lambda i: (i, 0)),
            pl.BlockSpec((D, D), lambda i: (0, 0)),
            pl.BlockSpec((D, D), lambda i: (0, 0)),
            pl.BlockSpec((D, D), lambda i: (0, 0)),
        ],
        out_specs=[
            pl.BlockSpec((RT, D), lambda i: (i, 0)),
            pl.BlockSpec((RT, D), lambda i: (i, 0)),
            pl.BlockSpec((RT, D), lambda i: (i, 0)),
        ],
        out_shape=[jax.ShapeDtypeStruct((B * T, D), jnp.float32)] * 3,
        compiler_params=pltpu.CompilerParams(
            dimension_semantics=("parallel",),
            vmem_limit_bytes=52 * 1024 * 1024,
        ),
        name="titans_qkv_proj",
    )(x2, Wq.T, Wk.T, Wv.T)

    y2 = pl.pallas_call(
        _core_kernel,
        grid=(B * ngrp,),
        in_specs=[
            pl.BlockSpec((1, 2, _LH), lambda i: (i % ngrp, 0, 0)),
            pl.BlockSpec((T, _LH * DH), lambda i: (i // ngrp, i % ngrp)),
            pl.BlockSpec((T, _LH * DH), lambda i: (i // ngrp, i % ngrp)),
            pl.BlockSpec((T, _LH * DH), lambda i: (i // ngrp, i % ngrp)),
        ],
        out_specs=pl.BlockSpec((T, _LH * DH), lambda i: (i // ngrp, i % ngrp)),
        out_shape=jax.ShapeDtypeStruct((B * T, D), jnp.float32),
        scratch_shapes=[pltpu.VMEM((_LH, DH, DH), jnp.float32)],
        compiler_params=pltpu.CompilerParams(
            dimension_semantics=("parallel",),
            vmem_limit_bytes=52 * 1024 * 1024,
        ),
        name="titans_core",
    )(ab, q2, k2, v2)

    out2 = pl.pallas_call(
        _oproj_kernel,
        grid=(nrt,),
        in_specs=[
            pl.BlockSpec((RT, D), lambda i: (i, 0)),
            pl.BlockSpec((D, D), lambda i: (0, 0)),
        ],
        out_specs=pl.BlockSpec((RT, D), lambda i: (i, 0)),
        out_shape=jax.ShapeDtypeStruct((B * T, D), jnp.float32),
        compiler_params=pltpu.CompilerParams(
            dimension_semantics=("parallel",),
            vmem_limit_bytes=52 * 1024 * 1024,
        ),
        name="titans_out_proj",
    )(y2, Wo.T)

    return out2.reshape(B, T, D)


# speedup vs baseline: 7.5709x; 2.0053x over previous
import jax
import jax.numpy as jnp
from jax.experimental import pallas as pl
from jax.experimental.pallas import tpu as pltpu

_H = 16
_C = 64
_DH = 64
_G = 4
_GW = _G * _DH
_NG = _H // _G
_TT = 512
_SCALE = 0.125


def _proj_kernel(x_ref, wq_ref, wk_ref, wv_ref, q_ref, k_ref, v_ref):
    xt = x_ref[...]
    q_ref[...] = jnp.dot(xt, wq_ref[...], preferred_element_type=jnp.float32)
    k_ref[...] = jnp.dot(xt, wk_ref[...], preferred_element_type=jnp.float32)
    v_ref[...] = jnp.dot(xt, wv_ref[...], preferred_element_type=jnp.float32)


def _dotT(a, b):
    return jax.lax.dot_general(a, b, (((1,), (1,)), ((), ())),
                               preferred_element_type=jnp.float32)


def _dotTa(a, b):
    return jax.lax.dot_general(a, b, (((0,), (0,)), ((), ())),
                               preferred_element_type=jnp.float32)


def _dot(a, b):
    return jnp.dot(a, b, preferred_element_type=jnp.float32)


def _core_kernel(ab_ref, q_ref, k_ref, v_ref, o_ref, m_scr):
    C, DH, GW = _C, _DH, _GW

    rowg = jax.lax.broadcasted_iota(jnp.int32, (GW, GW), 0)
    colg = jax.lax.broadcasted_iota(jnp.int32, (GW, GW), 1)
    bdmask = (rowg // C) == (colg // C)
    umask = bdmask & ((colg % C) > (rowg % C))
    eye = jnp.where(rowg == colg, 1.0, 0.0).astype(jnp.float32)
    rows_s = jax.lax.broadcasted_iota(jnp.int32, (C, GW), 0)
    cols_s = jax.lax.broadcasted_iota(jnp.int32, (C, GW), 1)
    smask = (cols_s % C) > rows_s

    absig = jax.nn.sigmoid(ab_ref[...])

    @pl.when(pl.program_id(1) == 0)
    def _():
        m_scr[...] = jnp.zeros_like(m_scr)

    def chunk_body(n, carry):
        r0 = pl.multiple_of(n * C, C)
        rows = pl.ds(r0, C)
        for g in range(_NG):
            cols = slice(g * GW, (g + 1) * GW)
            avec = absig[0:1, cols] * 0.5
            bvec = absig[1:2, cols] * 2.0

            q = q_ref[rows, cols]
            k = k_ref[rows, cols]
            v = v_ref[rows, cols]

            kk = k * k
            kn_parts = []
            for j in range(_G):
                cj = slice(j * DH, (j + 1) * DH)
                s = jnp.sum(kk[:, cj], axis=1, keepdims=True)
                kn_parts.append(k[:, cj] /
                                jnp.maximum(jnp.sqrt(s), 1e-12))
            kn = jnp.concatenate(kn_parts, axis=1)

            kn4 = jnp.concatenate([kn, kn, kn, kn], axis=0)
            Kbd = jnp.where(bdmask, kn4, 0.0)
            v4 = jnp.concatenate([v, v, v, v], axis=0)
            Vbd = jnp.where(bdmask, v4, 0.0)

            M = m_scr[g]

            scores = _dotT(q, Kbd) * _SCALE
            scores = jnp.where(smask, -1e30, scores)
            attn_parts = []
            for j in range(_G):
                cj = slice(j * DH, (j + 1) * DH)
                sc = scores[:, cj]
                m = jnp.max(sc, axis=1, keepdims=True)
                e = jnp.exp(sc - m)
                attn_parts.append(e / jnp.sum(e, axis=1, keepdims=True))
            attn = jnp.concatenate(attn_parts, axis=1)

            attn_out = _dot(attn, Vbd)
            mem_out = _dot(q, M)
            o_ref[rows, cols] = attn_out + 0.1 * mem_out

            G = _dotT(Kbd, Kbd)
            aU = jnp.where(umask, G, 0.0) * avec
            inv = eye - aU
            npow = aU
            for _ in range(5):
                npow = _dot(npow, npow)
                inv = inv + _dot(inv, npow)
            R = _dot(inv, Kbd)
            KtRa = _dotTa(Kbd, R * avec)
            T1 = _dot(M, KtRa)
            T2 = _dotTa(Vbd, R * bvec)
            m_scr[g] = M - T1 + T2
        return carry

    jax.lax.fori_loop(0, _TT // C, chunk_body, 0)


def _oproj_kernel(y_ref, w_ref, o_ref):
    o_ref[...] = jnp.dot(y_ref[...], w_ref[...],
                         preferred_element_type=jnp.float32)


def kernel(x, Wq, Wk, Wv, Wo, alpha_raw, beta_raw):
    B, T, D = x.shape
    H = _H
    x2 = x.reshape(B * T, D)
    ab = jnp.repeat(
        jnp.concatenate([alpha_raw.reshape(1, H), beta_raw.reshape(1, H)],
                        axis=0), D // H, axis=1)

    RT = 512
    nrt = (B * T) // RT
    q2, k2, v2 = pl.pallas_call(
        _proj_kernel,
        grid=(nrt,),
        in_specs=[
            pl.BlockSpec((RT, D), lambda i: (i, 0)),
            pl.BlockSpec((D, D), lambda i: (0, 0)),
            pl.BlockSpec((D, D), lambda i: (0, 0)),
            pl.BlockSpec((D, D), lambda i: (0, 0)),
        ],
        out_specs=[
            pl.BlockSpec((RT, D), lambda i: (i, 0)),
            pl.BlockSpec((RT, D), lambda i: (i, 0)),
            pl.BlockSpec((RT, D), lambda i: (i, 0)),
        ],
        out_shape=[jax.ShapeDtypeStruct((B * T, D), jnp.float32)] * 3,
        compiler_params=pltpu.CompilerParams(
            dimension_semantics=("parallel",),
            vmem_limit_bytes=52 * 1024 * 1024,
        ),
        name="titans_qkv_proj",
    )(x2, Wq.T, Wk.T, Wv.T)

    ntt = T // _TT
    y2 = pl.pallas_call(
        _core_kernel,
        grid=(B, ntt),
        in_specs=[
            pl.BlockSpec((2, D), lambda b, t: (0, 0)),
            pl.BlockSpec((_TT, D), lambda b, t: (b * ntt + t, 0)),
            pl.BlockSpec((_TT, D), lambda b, t: (b * ntt + t, 0)),
            pl.BlockSpec((_TT, D), lambda b, t: (b * ntt + t, 0)),
        ],
        out_specs=pl.BlockSpec((_TT, D), lambda b, t: (b * ntt + t, 0)),
        out_shape=jax.ShapeDtypeStruct((B * T, D), jnp.float32),
        scratch_shapes=[pltpu.VMEM((_NG, _GW, _GW), jnp.float32)],
        compiler_params=pltpu.CompilerParams(
            dimension_semantics=("parallel", "arbitrary"),
            vmem_limit_bytes=52 * 1024 * 1024,
        ),
        name="titans_core",
    )(ab, q2, k2, v2)

    out2 = pl.pallas_call(
        _oproj_kernel,
        grid=(nrt,),
        in_specs=[
            pl.BlockSpec((RT, D), lambda i: (i, 0)),
            pl.BlockSpec((D, D), lambda i: (0, 0)),
        ],
        out_specs=pl.BlockSpec((RT, D), lambda i: (i, 0)),
        out_shape=jax.ShapeDtypeStruct((B * T, D), jnp.float32),
        compiler_params=pltpu.CompilerParams(
            dimension_semantics=("parallel",),
            vmem_limit_bytes=52 * 1024 * 1024,
        ),
        name="titans_out_proj",
    )(y2, Wo.T)

    return out2.reshape(B, T, D)


# speedup vs baseline: 10.9839x; 1.4508x over previous
import jax
import jax.numpy as jnp
import numpy as np
from jax.experimental import pallas as pl
from jax.experimental.pallas import tpu as pltpu
try:
    from jax.experimental.shard_map import shard_map
except ImportError:
    from jax.sharding import shard_map

_H = 16
_C = 64
_DH = 64
_G = 4
_GW = _G * _DH
_NG = _H // _G
_TT = 512
_SCALE = 0.125


def _proj_kernel(x_ref, wq_ref, wk_ref, wv_ref, q_ref, k_ref, v_ref):
    xt = x_ref[...]
    q_ref[...] = jnp.dot(xt, wq_ref[...], preferred_element_type=jnp.float32)
    k_ref[...] = jnp.dot(xt, wk_ref[...], preferred_element_type=jnp.float32)
    v_ref[...] = jnp.dot(xt, wv_ref[...], preferred_element_type=jnp.float32)


def _dotT(a, b):
    return jax.lax.dot_general(a, b, (((1,), (1,)), ((), ())),
                               preferred_element_type=jnp.float32)


def _dotTa(a, b):
    return jax.lax.dot_general(a, b, (((0,), (0,)), ((), ())),
                               preferred_element_type=jnp.float32)


def _dot(a, b):
    return jnp.dot(a, b, preferred_element_type=jnp.float32)


def _core_kernel(ab_ref, q_ref, k_ref, v_ref, o_ref, m_scr):
    C, DH, GW = _C, _DH, _GW

    rowg = jax.lax.broadcasted_iota(jnp.int32, (GW, GW), 0)
    colg = jax.lax.broadcasted_iota(jnp.int32, (GW, GW), 1)
    bdmask = (rowg // C) == (colg // C)
    umask = bdmask & ((colg % C) > (rowg % C))
    eye = jnp.where(rowg == colg, 1.0, 0.0).astype(jnp.float32)
    rows_s = jax.lax.broadcasted_iota(jnp.int32, (C, GW), 0)
    cols_s = jax.lax.broadcasted_iota(jnp.int32, (C, GW), 1)
    smask = (cols_s % C) > rows_s

    absig = jax.nn.sigmoid(ab_ref[...])

    @pl.when(pl.program_id(2) == 0)
    def _():
        m_scr[...] = jnp.zeros_like(m_scr)

    def chunk_body(n, carry):
        r0 = pl.multiple_of(n * C, C)
        rows = pl.ds(r0, C)
        for g in range(_NG):
            cols = slice(g * GW, (g + 1) * GW)
            avec = absig[0:1, cols] * 0.5
            bvec = absig[1:2, cols] * 2.0

            q = q_ref[rows, cols]
            k = k_ref[rows, cols]
            v = v_ref[rows, cols]

            kk = k * k
            kn_parts = []
            for j in range(_G):
                cj = slice(j * DH, (j + 1) * DH)
                s = jnp.sum(kk[:, cj], axis=1, keepdims=True)
                kn_parts.append(k[:, cj] /
                                jnp.maximum(jnp.sqrt(s), 1e-12))
            kn = jnp.concatenate(kn_parts, axis=1)

            kn4 = jnp.concatenate([kn, kn, kn, kn], axis=0)
            Kbd = jnp.where(bdmask, kn4, 0.0)
            v4 = jnp.concatenate([v, v, v, v], axis=0)
            Vbd = jnp.where(bdmask, v4, 0.0)

            M = m_scr[g]

            scores = _dotT(q, Kbd) * _SCALE
            scores = jnp.where(smask, -1e30, scores)
            attn_parts = []
            for j in range(_G):
                cj = slice(j * DH, (j + 1) * DH)
                sc = scores[:, cj]
                m = jnp.max(sc, axis=1, keepdims=True)
                e = jnp.exp(sc - m)
                attn_parts.append(e / jnp.sum(e, axis=1, keepdims=True))
            attn = jnp.concatenate(attn_parts, axis=1)

            attn_out = _dot(attn, Vbd)
            mem_out = _dot(q, M)
            o_ref[rows, cols] = attn_out + 0.1 * mem_out

            G = _dotT(Kbd, Kbd)
            aU = jnp.where(umask, G, 0.0) * avec
            inv = eye - aU
            npow = aU
            for _ in range(5):
                npow = _dot(npow, npow)
                inv = inv + _dot(inv, npow)
            R = _dot(inv, Kbd)
            KtRa = _dotTa(Kbd, R * avec)
            T1 = _dot(M, KtRa)
            T2 = _dotTa(Vbd, R * bvec)
            m_scr[g] = M - T1 + T2
        return carry

    jax.lax.fori_loop(0, _TT // C, chunk_body, 0)


def _oproj_kernel(y_ref, w_ref, o_ref):
    o_ref[...] = jnp.dot(y_ref[...], w_ref[...],
                         preferred_element_type=jnp.float32)


def _impl(x, Wq, Wk, Wv, Wo, alpha_raw, beta_raw):
    B, T, D = x.shape
    H = _H
    x2 = x.reshape(B * T, D)
    ab = jnp.repeat(
        jnp.concatenate([alpha_raw.reshape(1, H), beta_raw.reshape(1, H)],
                        axis=0), D // H, axis=1)

    RT = 512
    nrt = (B * T) // RT
    nrt2 = nrt // 2
    q2, k2, v2 = pl.pallas_call(
        _proj_kernel,
        grid=(2, nrt2),
        in_specs=[
            pl.BlockSpec((RT, D), lambda c, i: (c * nrt2 + i, 0)),
            pl.BlockSpec((D, D), lambda c, i: (0, 0)),
            pl.BlockSpec((D, D), lambda c, i: (0, 0)),
            pl.BlockSpec((D, D), lambda c, i: (0, 0)),
        ],
        out_specs=[
            pl.BlockSpec((RT, D), lambda c, i: (c * nrt2 + i, 0)),
            pl.BlockSpec((RT, D), lambda c, i: (c * nrt2 + i, 0)),
            pl.BlockSpec((RT, D), lambda c, i: (c * nrt2 + i, 0)),
        ],
        out_shape=[jax.ShapeDtypeStruct((B * T, D), jnp.float32)] * 3,
        compiler_params=pltpu.CompilerParams(
            dimension_semantics=("parallel", "arbitrary"),
            vmem_limit_bytes=52 * 1024 * 1024,
        ),
        name="titans_qkv_proj",
    )(x2, Wq.T, Wk.T, Wv.T)

    ntt = T // _TT
    bhalf = B // 2

    def _cidx(c, b, t):
        return ((c * bhalf + b) * ntt + t, 0)

    y2 = pl.pallas_call(
        _core_kernel,
        grid=(2, bhalf, ntt),
        in_specs=[
            pl.BlockSpec((2, D), lambda c, b, t: (0, 0)),
            pl.BlockSpec((_TT, D), _cidx),
            pl.BlockSpec((_TT, D), _cidx),
            pl.BlockSpec((_TT, D), _cidx),
        ],
        out_specs=pl.BlockSpec((_TT, D), _cidx),
        out_shape=jax.ShapeDtypeStruct((B * T, D), jnp.float32),
        scratch_shapes=[pltpu.VMEM((_NG, _GW, _GW), jnp.float32)],
        compiler_params=pltpu.CompilerParams(
            dimension_semantics=("parallel", "arbitrary", "arbitrary"),
            vmem_limit_bytes=52 * 1024 * 1024,
        ),
        name="titans_core",
    )(ab, q2, k2, v2)

    out2 = pl.pallas_call(
        _oproj_kernel,
        grid=(2, nrt2),
        in_specs=[
            pl.BlockSpec((RT, D), lambda c, i: (c * nrt2 + i, 0)),
            pl.BlockSpec((D, D), lambda c, i: (0, 0)),
        ],
        out_specs=pl.BlockSpec((RT, D), lambda c, i: (c * nrt2 + i, 0)),
        out_shape=jax.ShapeDtypeStruct((B * T, D), jnp.float32),
        compiler_params=pltpu.CompilerParams(
            dimension_semantics=("parallel", "arbitrary"),
            vmem_limit_bytes=52 * 1024 * 1024,
        ),
        name="titans_out_proj",
    )(y2, Wo.T)

    return out2.reshape(B, T, D)


def kernel(x, Wq, Wk, Wv, Wo, alpha_raw, beta_raw):
    devs = jax.devices()
    B = x.shape[0]
    if len(devs) >= 2 and B % 2 == 0:
        mesh = jax.sharding.Mesh(np.array(devs[:2]), ("b",))
        P = jax.sharding.PartitionSpec
        f = shard_map(
            _impl, mesh=mesh,
            in_specs=(P("b"), P(), P(), P(), P(), P(), P()),
            out_specs=P("b"), check_rep=False)
        return f(x, Wq, Wk, Wv, Wo, alpha_raw, beta_raw)
    return _impl(x, Wq, Wk, Wv, Wo, alpha_raw, beta_raw)
